# Initial kernel scaffold; baseline (speedup 1.0000x reference)
import jax
import jax.numpy as jnp
from jax.experimental import pallas as pl


def kernel(input, emb_weight, lin_w, lin_b):
    raise NotImplementedError("write your pallas kernel here")


# speedup vs baseline: 58.5265x; 58.5265x over previous
import functools

import jax
import jax.numpy as jnp
from jax import lax
from jax.experimental import pallas as pl
from jax.experimental.pallas import tpu as pltpu
from jax.experimental.pallas import tpu_sc as plsc

B = 16384
L = 200
V = 1000
D = 100
C = 21
CP = 32

NC, NS, LANES = 2, 16, 16
NW = NC * NS
BPW = B // NW
NBLK = B // LANES
BLK_PER_W = NBLK // NW
BLK_PER_SLAB = 16
NSLAB = BLK_PER_W // BLK_PER_SLAB


def _table_body(lin_w_ref, emb_t_ref, pt_ref):
    pt_ref[...] = lax.dot_general(
        lin_w_ref[...], emb_t_ref[...],
        (((1,), (0,)), ((), ())),
        preferred_element_type=jnp.float32,
        precision=lax.Precision.HIGHEST,
    ) * (1.0 / L)


def _make_table(lin_w_pad, emb_t):
    return pl.pallas_call(
        _table_body,
        out_shape=jax.ShapeDtypeStruct((CP, V), jnp.float32),
    )(lin_w_pad, emb_t)


_sc_mesh = plsc.VectorSubcoreMesh(
    core_axis_name="c", subcore_axis_name="s", num_cores=NC, num_subcores=NS)


@functools.partial(
    pl.kernel,
    out_type=jax.ShapeDtypeStruct((NW, C, BPW), jnp.float32),
    mesh=_sc_mesh,
    compiler_params=pltpu.CompilerParams(
        use_tc_tiling_on_sc=False, needs_layout_passes=False),
    scratch_types=[
        pltpu.VMEM((CP * V,), jnp.float32),
        pltpu.VMEM((C, LANES), jnp.float32),
        pltpu.VMEM((BLK_PER_SLAB, L, LANES), jnp.int32),
        pltpu.VMEM((C, BPW), jnp.float32),
    ],
)
def _sc_bag_kernel(pt_hbm, bias_hbm, idx_hbm, out_hbm,
                   pt_v, bias_v, slab_v, out_v):
    cid = lax.axis_index("c")
    sid = lax.axis_index("s")
    wid = sid * NC + cid
    pltpu.sync_copy(pt_hbm, pt_v)
    pltpu.sync_copy(bias_hbm, bias_v)
    blk_base = wid * BLK_PER_W
    for slab in range(NSLAB):
        pltpu.sync_copy(
            idx_hbm.at[pl.ds(blk_base + slab * BLK_PER_SLAB, BLK_PER_SLAB)],
            slab_v)
        for blk in range(BLK_PER_SLAB):
            acc0 = tuple(bias_v[c, :] for c in range(C))

            def body(l, accs, _blk=blk):
                idxv = slab_v[_blk, l, :]
                return tuple(
                    accs[c] + plsc.load_gather(pt_v, [idxv + (c * V)])
                    for c in range(C))

            accs = lax.fori_loop(0, L, body, acc0)
            col = (slab * BLK_PER_SLAB + blk) * LANES
            for c in range(C):
                out_v[c, pl.ds(col, LANES)] = accs[c]
    pltpu.sync_copy(out_v, out_hbm.at[wid])


def kernel(input, emb_weight, lin_w, lin_b):
    idx = input.astype(jnp.int32)
    idx3 = idx.reshape(NBLK, LANES, L).transpose(0, 2, 1)
    lin_w_pad = jnp.zeros((CP, D), jnp.float32).at[:C].set(lin_w)
    pt = _make_table(lin_w_pad, emb_weight.T).reshape(CP * V)
    bias2 = jnp.broadcast_to(lin_b[:, None], (C, LANES))
    out3 = _sc_bag_kernel(pt, bias2, idx3)
    return out3.transpose(0, 2, 1).reshape(B, C)


# speedup vs baseline: 77.1205x; 1.3177x over previous
import functools

import jax
import jax.numpy as jnp
from jax import lax
from jax.experimental import pallas as pl
from jax.experimental.pallas import tpu as pltpu
from jax.experimental.pallas import tpu_sc as plsc

B = 16384
L = 200
V = 1000
D = 100
C = 21
CP = 32
NPAIR = 11

NC, NS, LANES = 2, 16, 16
NW = NC * NS
BPW = B // NW
BLK_PER_W = BPW // LANES
BLK_PER_SLAB = 16
NSLAB = BLK_PER_W // BLK_PER_SLAB
SLAB_WORDS = BLK_PER_SLAB * LANES * L
CHUNK = 25
NCHUNK = L // CHUNK


def _table_body(lin_w_ref, emb_t_ref, pt_ref):
    pt_ref[...] = lax.dot_general(
        lin_w_ref[...], emb_t_ref[...],
        (((1,), (0,)), ((), ())),
        preferred_element_type=jnp.float32,
        precision=lax.Precision.HIGHEST,
    ) * (1.0 / L)


def _make_table(lin_w_pad, emb_t):
    return pl.pallas_call(
        _table_body,
        out_shape=jax.ShapeDtypeStruct((CP, V), jnp.float32),
    )(lin_w_pad, emb_t)


_sc_mesh = plsc.VectorSubcoreMesh(
    core_axis_name="c", subcore_axis_name="s", num_cores=NC, num_subcores=NS)


@functools.partial(
    pl.kernel,
    out_type=jax.ShapeDtypeStruct((B, C), jnp.float32),
    mesh=_sc_mesh,
    compiler_params=pltpu.CompilerParams(
        use_tc_tiling_on_sc=False, needs_layout_passes=False),
    scratch_types=[
        pltpu.VMEM((NPAIR * V,), jnp.int32),
        pltpu.VMEM((2 * NPAIR, LANES), jnp.float32),
        pltpu.VMEM((SLAB_WORDS,), jnp.int32),
        pltpu.VMEM((L * LANES,), jnp.int32),
        pltpu.VMEM((BPW, C), jnp.float32),
    ],
)
def _sc_bag_kernel(pk_hbm, bias_hbm, idx_hbm, out_hbm,
                   pk_v, bias_v, slab_v, idxt_v, out_v):
    cid = lax.axis_index("c")
    sid = lax.axis_index("s")
    wid = sid * NC + cid
    pltpu.sync_copy(pk_hbm, pk_v)
    pltpu.sync_copy(bias_hbm, bias_v)
    iota = lax.iota(jnp.int32, LANES)
    zero_pair = jnp.zeros((2 * LANES,), jnp.bfloat16)

    for slab in range(NSLAB):
        pltpu.sync_copy(
            idx_hbm.at[pl.ds((wid * NSLAB + slab) * SLAB_WORDS, SLAB_WORDS)],
            slab_v)

        def block_body(blk, _, _slab=slab):
            def tr_body(v, carry):
                src = slab_v[pl.ds(blk * (LANES * L) + v * LANES, LANES)]
                w = v * LANES + iota
                bb = w // L
                ll = w - bb * L
                plsc.store_scatter(idxt_v, [ll * LANES + bb], src)
                return carry

            lax.fori_loop(0, L, tr_body, 0)

            acc_f = [bias_v[c, :] for c in range(2 * NPAIR)]
            for chunk in range(NCHUNK):
                def ch_body(i, accs, _chunk=chunk):
                    idxv = idxt_v[pl.ds((_chunk * CHUNK + i) * LANES, LANES)]
                    return tuple(
                        accs[p] + plsc.bitcast(
                            plsc.load_gather(pk_v, [idxv + (p * V)]),
                            jnp.bfloat16)
                        for p in range(NPAIR))

                acc_b = lax.fori_loop(
                    0, CHUNK, ch_body, (zero_pair,) * NPAIR)
                for p in range(NPAIR):
                    lo, hi = plsc.unpack(
                        acc_b[p], format=plsc.PackFormat.INTERLEAVED)
                    acc_f[2 * p] = acc_f[2 * p] + lo
                    acc_f[2 * p + 1] = acc_f[2 * p + 1] + hi

            rowv = (_slab * BLK_PER_SLAB + blk) * LANES + iota
            for c in range(C):
                plsc.store_scatter(
                    out_v, [rowv, jnp.full((LANES,), c, jnp.int32)],
                    acc_f[c])
            return 0

        lax.fori_loop(0, BLK_PER_SLAB, block_body, 0)

    pltpu.sync_copy(out_v, out_hbm.at[pl.ds(wid * BPW, BPW)])


def kernel(input, emb_weight, lin_w, lin_b):
    idx_flat = input.astype(jnp.int32).reshape(B * L)
    lin_w_pad = jnp.zeros((CP, D), jnp.float32).at[:C].set(lin_w)
    pt = _make_table(lin_w_pad, emb_weight.T)
    pe = pt[0:2 * NPAIR:2]
    po = pt[1:2 * NPAIR:2]
    ue = lax.bitcast_convert_type(
        pe.astype(jnp.bfloat16), jnp.uint16).astype(jnp.uint32)
    uo = lax.bitcast_convert_type(
        po.astype(jnp.bfloat16), jnp.uint16).astype(jnp.uint32)
    pk = lax.bitcast_convert_type(
        ue | (uo << 16), jnp.int32).reshape(NPAIR * V)
    bias2 = jnp.zeros((2 * NPAIR, LANES), jnp.float32).at[:C].set(
        jnp.broadcast_to(lin_b[:, None], (C, LANES)))
    return _sc_bag_kernel(pk, bias2, idx_flat)


# speedup vs baseline: 79.1195x; 1.0259x over previous
import functools

import jax
import jax.numpy as jnp
from jax import lax
from jax.experimental import pallas as pl
from jax.experimental.pallas import tpu as pltpu
from jax.experimental.pallas import tpu_sc as plsc

B = 16384
L = 200
V = 1000
D = 100
C = 21
NPAIR = 11

NC, NS, LANES = 2, 16, 16
NW = NC * NS
BPW = B // NW
BLK_PER_W = BPW // LANES
BLK_PER_SLAB = 16
NSLAB = BLK_PER_W // BLK_PER_SLAB
SLAB_WORDS = BLK_PER_SLAB * LANES * L
CHUNK = 25
NCHUNK = L // CHUNK


def _table_body(lin_w_ref, emb_ref, lin_b_ref, pk_ref):
    p = lax.dot_general(
        lin_w_ref[...], emb_ref[...],
        (((1,), (1,)), ((), ())),
        preferred_element_type=jnp.float32,
        precision=lax.Precision.HIGHEST,
    ) * (1.0 / L) + lin_b_ref[...] * (1.0 / L)
    lo = p[:NPAIR, :]
    hi = jnp.concatenate(
        [p[NPAIR:, :], jnp.zeros((1, V), jnp.float32)], axis=0)
    ulo = lax.bitcast_convert_type(
        lo.astype(jnp.bfloat16), jnp.uint16).astype(jnp.uint32)
    uhi = lax.bitcast_convert_type(
        hi.astype(jnp.bfloat16), jnp.uint16).astype(jnp.uint32)
    pk_ref[...] = lax.bitcast_convert_type(ulo | (uhi << 16), jnp.int32)


def _make_table(lin_w, emb_weight, lin_b_col):
    return pl.pallas_call(
        _table_body,
        out_shape=jax.ShapeDtypeStruct((NPAIR, V), jnp.int32),
    )(lin_w, emb_weight, lin_b_col)


_sc_mesh = plsc.VectorSubcoreMesh(
    core_axis_name="c", subcore_axis_name="s", num_cores=NC, num_subcores=NS)


@functools.partial(
    pl.kernel,
    out_type=jax.ShapeDtypeStruct((B, C), jnp.float32),
    mesh=_sc_mesh,
    compiler_params=pltpu.CompilerParams(
        use_tc_tiling_on_sc=False, needs_layout_passes=False),
    scratch_types=[
        pltpu.VMEM((NPAIR * V,), jnp.int32),
        pltpu.VMEM((SLAB_WORDS,), jnp.int32),
        pltpu.VMEM((L * LANES,), jnp.int32),
        pltpu.VMEM((BPW, C), jnp.float32),
    ],
)
def _sc_bag_kernel(pk_hbm, idx_hbm, out_hbm, pk_v, slab_v, idxt_v, out_v):
    cid = lax.axis_index("c")
    sid = lax.axis_index("s")
    wid = sid * NC + cid
    pltpu.sync_copy(pk_hbm, pk_v)
    iota = lax.iota(jnp.int32, LANES)
    zero_pair = jnp.zeros((2 * LANES,), jnp.bfloat16)
    zero_f = jnp.zeros((LANES,), jnp.float32)

    for slab in range(NSLAB):
        pltpu.sync_copy(
            idx_hbm.at[pl.ds((wid * NSLAB + slab) * SLAB_WORDS, SLAB_WORDS)],
            slab_v)

        def block_body(blk, _, _slab=slab):
            def tr_body(v, carry):
                src = slab_v[pl.ds(blk * (LANES * L) + v * LANES, LANES)]
                w = v * LANES + iota
                bb = w // L
                ll = w - bb * L
                plsc.store_scatter(idxt_v, [ll * LANES + bb], src)
                return carry

            lax.fori_loop(0, L, tr_body, 0)

            acc_f = [zero_f] * (2 * NPAIR)
            for chunk in range(NCHUNK):
                def ch_body(i, accs, _chunk=chunk):
                    idxv = idxt_v[pl.ds((_chunk * CHUNK + i) * LANES, LANES)]
                    return tuple(
                        accs[p] + plsc.bitcast(
                            plsc.load_gather(pk_v, [idxv + (p * V)]),
                            jnp.bfloat16)
                        for p in range(NPAIR))

                acc_b = lax.fori_loop(
                    0, CHUNK, ch_body, (zero_pair,) * NPAIR)
                for p in range(NPAIR):
                    lo, hi = plsc.unpack(
                        acc_b[p], format=plsc.PackFormat.INTERLEAVED)
                    acc_f[p] = acc_f[p] + lo
                    acc_f[p + NPAIR] = acc_f[p + NPAIR] + hi

            rowv = (_slab * BLK_PER_SLAB + blk) * LANES + iota
            for c in range(C):
                plsc.store_scatter(
                    out_v, [rowv, jnp.full((LANES,), c, jnp.int32)],
                    acc_f[c])
            return 0

        lax.fori_loop(0, BLK_PER_SLAB, block_body, 0)

    pltpu.sync_copy(out_v, out_hbm.at[pl.ds(wid * BPW, BPW)])


def kernel(input, emb_weight, lin_w, lin_b):
    idx_flat = input.astype(jnp.int32).reshape(B * L)
    pk = _make_table(lin_w, emb_weight, lin_b.reshape(C, 1))
    return _sc_bag_kernel(pk.reshape(NPAIR * V), idx_flat)


# speedup vs baseline: 93.6275x; 1.1834x over previous
import functools

import jax
import jax.numpy as jnp
from jax import lax
from jax.experimental import pallas as pl
from jax.experimental.pallas import tpu as pltpu
from jax.experimental.pallas import tpu_sc as plsc

B = 16384
L = 200
V = 1000
D = 100
C = 21
NPAIR = 11

NC, NS, LANES = 2, 16, 16
NW = NC * NS
BPW = B // NW
BLK_PER_W = BPW // LANES
BLK_PER_SLAB = 16
NSLAB = BLK_PER_W // BLK_PER_SLAB
SLAB_WORDS = BLK_PER_SLAB * LANES * L
CHUNK = 25
NCHUNK = L // CHUNK


def _table_body(lin_w_ref, emb_ref, lin_b_ref, pk_ref):
    p = lax.dot_general(
        lin_w_ref[...], emb_ref[...],
        (((1,), (1,)), ((), ())),
        preferred_element_type=jnp.float32,
        precision=lax.Precision.HIGHEST,
    ) * (1.0 / L) + lin_b_ref[...] * (1.0 / L)
    lo = p[:NPAIR, :]
    hi = jnp.concatenate(
        [p[NPAIR:, :], jnp.zeros((1, V), jnp.float32)], axis=0)
    ulo = lax.bitcast_convert_type(
        lo.astype(jnp.bfloat16), jnp.uint16).astype(jnp.uint32)
    uhi = lax.bitcast_convert_type(
        hi.astype(jnp.bfloat16), jnp.uint16).astype(jnp.uint32)
    pk_ref[...] = lax.bitcast_convert_type(ulo | (uhi << 16), jnp.int32)


def _make_table(lin_w, emb_weight, lin_b_col):
    return pl.pallas_call(
        _table_body,
        out_shape=jax.ShapeDtypeStruct((NPAIR, V), jnp.int32),
    )(lin_w, emb_weight, lin_b_col)


_sc_mesh = plsc.VectorSubcoreMesh(
    core_axis_name="c", subcore_axis_name="s", num_cores=NC, num_subcores=NS)


@functools.partial(
    pl.kernel,
    out_type=jax.ShapeDtypeStruct((B, C), jnp.float32),
    mesh=_sc_mesh,
    compiler_params=pltpu.CompilerParams(
        use_tc_tiling_on_sc=False, needs_layout_passes=False),
    scratch_types=[
        pltpu.VMEM((NPAIR * V,), jnp.int32),
        pltpu.VMEM((2 * SLAB_WORDS,), jnp.int32),
        pltpu.VMEM((BPW, C), jnp.float32),
        pltpu.SemaphoreType.DMA,
        pltpu.SemaphoreType.DMA,
    ],
)
def _sc_bag_kernel(pk_hbm, idx_hbm, out_hbm, pk_v, slab_v, out_v,
                   sem0, sem1):
    cid = lax.axis_index("c")
    sid = lax.axis_index("s")
    wid = sid * NC + cid
    iota = lax.iota(jnp.int32, LANES)
    iota_l = iota * L
    zero_pair = jnp.zeros((2 * LANES,), jnp.bfloat16)
    zero_f = jnp.zeros((LANES,), jnp.float32)
    sems = (sem0, sem1)

    copies = [
        pltpu.async_copy(
            idx_hbm.at[pl.ds((wid * NSLAB + s) * SLAB_WORDS, SLAB_WORDS)],
            slab_v.at[pl.ds(s * SLAB_WORDS, SLAB_WORDS)],
            sems[s])
        for s in range(NSLAB)
    ]
    pltpu.sync_copy(pk_hbm, pk_v)

    for slab in range(NSLAB):
        copies[slab].wait()

        def block_body(blk, _, _slab=slab):
            base0 = (_slab * SLAB_WORDS + blk * (LANES * L))
            acc_f = [zero_f] * (2 * NPAIR)
            for chunk in range(NCHUNK):
                def ch_body(i, accs, _chunk=chunk):
                    pos = base0 + (_chunk * CHUNK + i)
                    idxv = plsc.load_gather(slab_v, [iota_l + pos])
                    return tuple(
                        accs[p] + plsc.bitcast(
                            plsc.load_gather(pk_v, [idxv + (p * V)]),
                            jnp.bfloat16)
                        for p in range(NPAIR))

                acc_b = lax.fori_loop(
                    0, CHUNK, ch_body, (zero_pair,) * NPAIR)
                for p in range(NPAIR):
                    lo, hi = plsc.unpack(
                        acc_b[p], format=plsc.PackFormat.INTERLEAVED)
                    acc_f[p] = acc_f[p] + lo
                    acc_f[p + NPAIR] = acc_f[p + NPAIR] + hi

            rowv = (_slab * BLK_PER_SLAB + blk) * LANES + iota
            for c in range(C):
                plsc.store_scatter(
                    out_v, [rowv, jnp.full((LANES,), c, jnp.int32)],
                    acc_f[c])
            return 0

        lax.fori_loop(0, BLK_PER_SLAB, block_body, 0)

    pltpu.sync_copy(out_v, out_hbm.at[pl.ds(wid * BPW, BPW)])


def kernel(input, emb_weight, lin_w, lin_b):
    idx_flat = input.astype(jnp.int32).reshape(B * L)
    pk = _make_table(lin_w, emb_weight, lin_b.reshape(C, 1))
    return _sc_bag_kernel(pk.reshape(NPAIR * V), idx_flat)


# speedup vs baseline: 96.3862x; 1.0295x over previous
import functools

import jax
import jax.numpy as jnp
from jax import lax
from jax.experimental import pallas as pl
from jax.experimental.pallas import tpu as pltpu
from jax.experimental.pallas import tpu_sc as plsc

B = 16384
L = 200
V = 1000
D = 100
C = 21
NPAIR = 11

NC, NS, LANES = 2, 16, 16
NW = NC * NS
BPW = B // NW
BLK_PER_W = BPW // LANES
BLK_PER_SLAB = 16
NSLAB = BLK_PER_W // BLK_PER_SLAB
SLAB_WORDS = BLK_PER_SLAB * LANES * L
CHUNK = 25
NCHUNK = L // CHUNK


def _table_body(lin_w_ref, emb_ref, lin_b_ref, pk_ref):
    p = lax.dot_general(
        lin_w_ref[...], emb_ref[...],
        (((1,), (1,)), ((), ())),
        preferred_element_type=jnp.float32,
        precision=lax.Precision.HIGHEST,
    ) * (1.0 / L) + lin_b_ref[...] * (1.0 / L)
    lo = p[:NPAIR, :]
    hi = jnp.concatenate(
        [p[NPAIR:, :], jnp.zeros((1, V), jnp.float32)], axis=0)
    ulo = lax.bitcast_convert_type(
        lo.astype(jnp.bfloat16), jnp.uint16).astype(jnp.uint32)
    uhi = lax.bitcast_convert_type(
        hi.astype(jnp.bfloat16), jnp.uint16).astype(jnp.uint32)
    pk_ref[...] = lax.bitcast_convert_type(ulo | (uhi << 16), jnp.int32)


def _make_table(lin_w, emb_weight, lin_b_col):
    return pl.pallas_call(
        _table_body,
        out_shape=jax.ShapeDtypeStruct((NPAIR, V), jnp.int32),
    )(lin_w, emb_weight, lin_b_col)


_sc_mesh = plsc.VectorSubcoreMesh(
    core_axis_name="c", subcore_axis_name="s", num_cores=NC, num_subcores=NS)


@functools.partial(
    pl.kernel,
    out_type=jax.ShapeDtypeStruct((B, C), jnp.float32),
    mesh=_sc_mesh,
    compiler_params=pltpu.CompilerParams(
        use_tc_tiling_on_sc=False, needs_layout_passes=False),
    scratch_types=[
        pltpu.VMEM((NPAIR * V,), jnp.int32),
        pltpu.VMEM((2 * SLAB_WORDS,), jnp.int32),
        pltpu.VMEM((BPW, C), jnp.float32),
        pltpu.SemaphoreType.DMA,
        pltpu.SemaphoreType.DMA,
    ],
)
def _sc_bag_kernel(pk_hbm, idx_hbm, out_hbm, pk_v, slab_v, out_v,
                   sem0, sem1):
    cid = lax.axis_index("c")
    sid = lax.axis_index("s")
    wid = sid * NC + cid
    iota = lax.iota(jnp.int32, LANES)
    iota_l = iota * L
    zero_pair = jnp.zeros((2 * LANES,), jnp.bfloat16)
    zero_f = jnp.zeros((LANES,), jnp.float32)
    sems = (sem0, sem1)

    copies = [
        pltpu.async_copy(
            idx_hbm.at[pl.ds((wid * NSLAB + s) * SLAB_WORDS, SLAB_WORDS)],
            slab_v.at[pl.ds(s * SLAB_WORDS, SLAB_WORDS)],
            sems[s])
        for s in range(NSLAB)
    ]
    pltpu.sync_copy(pk_hbm, pk_v)

    for slab in range(NSLAB):
        copies[slab].wait()

        def block_body(blk, _, _slab=slab):
            base0 = (_slab * SLAB_WORDS + blk * (LANES * L))
            acc_f = [zero_f] * (2 * NPAIR)
            for chunk in range(NCHUNK):
                def ch_body(i, accs, _chunk=chunk):
                    pos = base0 + (_chunk * CHUNK + i)
                    idxv = plsc.load_gather(slab_v, [iota_l + pos])
                    return tuple(
                        accs[p] + plsc.bitcast(
                            plsc.load_gather(pk_v, [idxv + (p * V)]),
                            jnp.bfloat16)
                        for p in range(NPAIR))

                acc_b = lax.fori_loop(
                    0, CHUNK, ch_body, (zero_pair,) * NPAIR, unroll=5)
                for p in range(NPAIR):
                    lo, hi = plsc.unpack(
                        acc_b[p], format=plsc.PackFormat.INTERLEAVED)
                    acc_f[p] = acc_f[p] + lo
                    acc_f[p + NPAIR] = acc_f[p + NPAIR] + hi

            rowv = (_slab * BLK_PER_SLAB + blk) * LANES + iota
            for c in range(C):
                plsc.store_scatter(
                    out_v, [rowv, jnp.full((LANES,), c, jnp.int32)],
                    acc_f[c])
            return 0

        lax.fori_loop(0, BLK_PER_SLAB, block_body, 0)

    pltpu.sync_copy(out_v, out_hbm.at[pl.ds(wid * BPW, BPW)])


def kernel(input, emb_weight, lin_w, lin_b):
    idx_flat = input.astype(jnp.int32).reshape(B * L)
    pk = _make_table(lin_w, emb_weight, lin_b.reshape(C, 1))
    return _sc_bag_kernel(pk.reshape(NPAIR * V), idx_flat)
